# baseline (speedup 1.0000x reference)
import jax
import jax.numpy as jnp
from jax.experimental import pallas as pl
from jax.experimental.pallas import tpu as pltpu

_ROWS_PER_BLOCK = 512


def _onehot_block(idx_ref, oh_ref, out_ref):
    idx = idx_ref[0, 0, :]
    r, ncls = out_ref.shape
    iota = jax.lax.broadcasted_iota(jnp.int32, (r, ncls), 1)
    base = oh_ref[0, :]
    out_ref[...] = jnp.where(iota == idx[:, None], 1.0, base)


def kernel(input, one_hot):
    orig = input.shape
    ncls = one_hot.shape[-1]
    data = input.reshape(-1).astype(jnp.int32)
    n = data.shape[0]
    r = _ROWS_PER_BLOCK
    nb = n // r
    data3 = data.reshape(nb, 1, r)
    out = pl.pallas_call(
        _onehot_block,
        grid=(nb,),
        in_specs=[
            pl.BlockSpec((1, 1, r), lambda i: (i, 0, 0)),
            pl.BlockSpec((1, ncls), lambda i: (0, 0)),
        ],
        out_specs=pl.BlockSpec((r, ncls), lambda i: (i, 0)),
        out_shape=jax.ShapeDtypeStruct((n, ncls), jnp.float32),
        compiler_params=pltpu.CompilerParams(
            dimension_semantics=("parallel",),
        ),
    )(data3, one_hot)
    return out.reshape(orig + (ncls,))


# speedup vs baseline: 1.4588x; 1.4588x over previous
import jax
import jax.numpy as jnp
from jax.experimental import pallas as pl
from jax.experimental.pallas import tpu as pltpu

_ROWS_PER_BLOCK = 64


def _onehot_block(idx_ref, oh_ref, out_ref):
    idx = idx_ref[...]
    b, s, ncls = out_ref.shape
    iota = jax.lax.broadcasted_iota(jnp.int32, (b, s, ncls), 2)
    base = oh_ref[0, :]
    out_ref[...] = jnp.where(iota == idx[:, :, None], 1.0, base)


def kernel(input, one_hot):
    rows, seq = input.shape
    ncls = one_hot.shape[-1]
    data = input.astype(jnp.int32)
    b = _ROWS_PER_BLOCK
    nb = rows // b
    out = pl.pallas_call(
        _onehot_block,
        grid=(nb,),
        in_specs=[
            pl.BlockSpec((b, seq), lambda i: (i, 0)),
            pl.BlockSpec((1, ncls), lambda i: (0, 0)),
        ],
        out_specs=pl.BlockSpec((b, seq, ncls), lambda i: (i, 0, 0)),
        out_shape=jax.ShapeDtypeStruct((rows, seq, ncls), jnp.float32),
    )(data, one_hot)
    return out
